# Initial kernel scaffold; baseline (speedup 1.0000x reference)
import functools

import jax
import jax.numpy as jnp
from jax import lax
from jax.experimental import pallas as pl
from jax.experimental.pallas import tpu as pltpu

H = 16
K = 2
QB = 512
BM = 256
FC = 1024
RB = 256


def _ln(x, scale, bias):
  mu = jnp.mean(x, -1, keepdims=True)
  var = jnp.mean((x - mu) ** 2, -1, keepdims=True)
  return (x - mu) * lax.rsqrt(var + 1e-5) * scale + bias


def _ln_qkv_body(x_ref, s_ref, b_ref, w_ref, bias_ref, o_ref):
  h = _ln(x_ref[...], s_ref[...], b_ref[...])
  o_ref[...] = (
      jnp.dot(h, w_ref[...], preferred_element_type=jnp.float32) + bias_ref[...]
  )


def _attn_body(q_ref, k_ref, v_ref, o_ref, *, T, dh):
  i = pl.program_id(2)
  q = q_ref[...]
  k = k_ref[...]
  v = v_ref[...]
  s = lax.dot_general(q, k, (((1,), (1,)), ((), ())),
                      preferred_element_type=jnp.float32)
  s = s * (1.0 / jnp.sqrt(jnp.float32(dh)))
  qrow = i * QB + lax.broadcasted_iota(jnp.int32, (QB, T), 0)
  kcol = lax.broadcasted_iota(jnp.int32, (QB, T), 1)
  s = jnp.where(qrow >= kcol, s, jnp.float32(-1e9))
  m = jnp.max(s, -1, keepdims=True)
  p = jnp.exp(s - m)
  p = p / jnp.sum(p, -1, keepdims=True)
  o_ref[...] = jnp.dot(p, v, preferred_element_type=jnp.float32)


def _proj_ln2_body(a_ref, x_ref, wo_ref, bo_ref, s2_ref, b2_ref, wg_ref,
                   ao_ref, mi_ref, lg_ref):
  a = (jnp.dot(a_ref[...], wo_ref[...], preferred_element_type=jnp.float32)
       + bo_ref[...] + x_ref[...])
  ao_ref[...] = a
  h = _ln(a, s2_ref[...], b2_ref[...])
  mi_ref[...] = h
  lg_ref[...] = jnp.dot(h, wg_ref[...], preferred_element_type=jnp.float32)


def _moe_body(meta_ref, x_ref, w1_ref, b1_ref, w2_ref, b2_ref, o_ref):
  s = pl.program_id(0)
  f = pl.program_id(1)
  blk = meta_ref[0, s]
  e = meta_ref[1, s]
  first = meta_ref[2, s]
  active = meta_ref[3, s]

  @pl.when((first == 1) & (f == 0))
  def _():
    o_ref[...] = jnp.zeros_like(o_ref)

  @pl.when(active == 1)
  def _():
    xb = x_ref[...]
    h = jax.nn.gelu(
        jnp.dot(xb, w1_ref[0], preferred_element_type=jnp.float32)
        + b1_ref[...])
    c = jnp.dot(h, w2_ref[0], preferred_element_type=jnp.float32)
    c = c + jnp.where(f == 0, 1.0, 0.0) * b2_ref[...]
    start = meta_ref[4, e]
    end = meta_ref[4, e + 1]
    row = blk * BM + lax.broadcasted_iota(jnp.int32, (BM, 1), 0)
    mask = (row >= start) & (row < end)
    o_ref[...] += jnp.where(mask, c, 0.0)


def kernel(x, ln1_scale, ln1_bias, ln2_scale, ln2_bias, Wqkv, bqkv, Wo, bo,
           Wg, W1, b1, W2, b2):
  B, T, D = x.shape
  E, _, F = W1.shape
  dh = D // H
  N = B * T
  NK = N * K
  NB = NK // BM
  S = NB + E - 1
  NF = F // FC

  x2d = x.reshape(N, D)
  ln1s = ln1_scale.reshape(1, D)
  ln1b = ln1_bias.reshape(1, D)
  ln2s = ln2_scale.reshape(1, D)
  ln2b = ln2_bias.reshape(1, D)

  qkv = pl.pallas_call(
      _ln_qkv_body,
      grid=(N // RB,),
      in_specs=[
          pl.BlockSpec((RB, D), lambda i: (i, 0)),
          pl.BlockSpec((1, D), lambda i: (0, 0)),
          pl.BlockSpec((1, D), lambda i: (0, 0)),
          pl.BlockSpec((D, 3 * D), lambda i: (0, 0)),
          pl.BlockSpec((1, 3 * D), lambda i: (0, 0)),
      ],
      out_specs=pl.BlockSpec((RB, 3 * D), lambda i: (i, 0)),
      out_shape=jax.ShapeDtypeStruct((N, 3 * D), jnp.float32),
  )(x2d, ln1s, ln1b, Wqkv, bqkv.reshape(1, 3 * D))

  nq = T // QB
  attn = pl.pallas_call(
      functools.partial(_attn_body, T=T, dh=dh),
      grid=(B, H, nq),
      in_specs=[
          pl.BlockSpec((QB, dh), lambda b, h, i: (b * (T // QB) + i, h)),
          pl.BlockSpec((T, dh), lambda b, h, i: (b, H + h)),
          pl.BlockSpec((T, dh), lambda b, h, i: (b, 2 * H + h)),
      ],
      out_specs=pl.BlockSpec((QB, dh), lambda b, h, i: (b * (T // QB) + i, h)),
      out_shape=jax.ShapeDtypeStruct((N, D), jnp.float32),
      compiler_params=pltpu.CompilerParams(
          dimension_semantics=("arbitrary", "arbitrary", "arbitrary")),
  )(qkv, qkv, qkv)

  EP = 128
  wg_p = jnp.zeros((D, EP), jnp.float32).at[:, :E].set(Wg)
  attn_out, moe_in, logits_p = pl.pallas_call(
      _proj_ln2_body,
      grid=(N // RB,),
      in_specs=[
          pl.BlockSpec((RB, D), lambda i: (i, 0)),
          pl.BlockSpec((RB, D), lambda i: (i, 0)),
          pl.BlockSpec((D, D), lambda i: (0, 0)),
          pl.BlockSpec((1, D), lambda i: (0, 0)),
          pl.BlockSpec((1, D), lambda i: (0, 0)),
          pl.BlockSpec((1, D), lambda i: (0, 0)),
          pl.BlockSpec((D, EP), lambda i: (0, 0)),
      ],
      out_specs=[
          pl.BlockSpec((RB, D), lambda i: (i, 0)),
          pl.BlockSpec((RB, D), lambda i: (i, 0)),
          pl.BlockSpec((RB, EP), lambda i: (i, 0)),
      ],
      out_shape=[
          jax.ShapeDtypeStruct((N, D), jnp.float32),
          jax.ShapeDtypeStruct((N, D), jnp.float32),
          jax.ShapeDtypeStruct((N, EP), jnp.float32),
      ],
  )(attn, x2d, Wo, bo.reshape(1, D), ln2s, ln2b, wg_p)

  logits = logits_p[:, :E]

  topv, topi = lax.top_k(logits, K)
  w = jax.nn.softmax(topv, axis=-1)
  flat_e = topi.reshape(-1).astype(jnp.int32)
  onehot = (flat_e[:, None] == jnp.arange(E, dtype=jnp.int32)).astype(jnp.int32)
  incl = jnp.cumsum(onehot, axis=0)
  counts = incl[-1]
  offsets = jnp.concatenate(
      [jnp.zeros((1,), jnp.int32), jnp.cumsum(counts).astype(jnp.int32)])
  rank = jnp.take_along_axis(incl, flat_e[:, None], 1)[:, 0] - 1
  inv = offsets[flat_e] + rank
  order = jnp.zeros((NK,), jnp.int32).at[inv].set(
      jnp.arange(NK, dtype=jnp.int32))
  tok_order = order // K

  start_e = offsets[:E]
  end_e = offsets[1:]
  nonempty = counts > 0
  firstb = jnp.where(nonempty, start_e // BM, 0)
  lastb = jnp.where(nonempty, (end_e - 1) // BM, -1)
  nb_e = jnp.where(nonempty, lastb - firstb + 1, 0).astype(jnp.int32)
  pos = jnp.concatenate(
      [jnp.zeros((1,), jnp.int32), jnp.cumsum(nb_e).astype(jnp.int32)])
  P = pos[E]
  pair_block = jnp.full((S + 1,), NB - 1, jnp.int32)
  pair_expert = jnp.zeros((S + 1,), jnp.int32)
  jblk = jnp.arange(NB, dtype=jnp.int32)
  for e in range(E):
    slots = jnp.where(jblk < nb_e[e], pos[e] + jblk, S)
    pair_block = pair_block.at[slots].set(firstb[e] + jblk)
    pair_expert = pair_expert.at[slots].set(e)
  pair_block = pair_block.at[S - 1].set(
      jnp.where(P < S, NB - 1, pair_block[S - 1]))
  pair_block = pair_block[:S]
  pair_expert = pair_expert[:S]
  pair_block = jnp.where(jnp.arange(S) < P, pair_block, NB - 1)
  first_flag = jnp.concatenate(
      [jnp.ones((1,), jnp.int32),
       (pair_block[1:] != pair_block[:-1]).astype(jnp.int32)])
  active_flag = (jnp.arange(S, dtype=jnp.int32) < P).astype(jnp.int32)
  meta = jnp.zeros((5, 64), jnp.int32)
  meta = meta.at[0, :S].set(pair_block)
  meta = meta.at[1, :S].set(pair_expert)
  meta = meta.at[2, :S].set(first_flag)
  meta = meta.at[3, :S].set(active_flag)
  meta = meta.at[4, : E + 1].set(offsets)

  gathered = moe_in[tok_order]

  out_sorted = pl.pallas_call(
      _moe_body,
      grid_spec=pltpu.PrefetchScalarGridSpec(
          num_scalar_prefetch=1,
          grid=(S, NF),
          in_specs=[
              pl.BlockSpec((BM, D), lambda s, f, m: (m[0, s], 0)),
              pl.BlockSpec((1, D, FC), lambda s, f, m: (m[1, s], 0, f)),
              pl.BlockSpec((1, FC), lambda s, f, m: (m[1, s], f)),
              pl.BlockSpec((1, FC, D), lambda s, f, m: (m[1, s], f, 0)),
              pl.BlockSpec((1, D), lambda s, f, m: (m[1, s], 0)),
          ],
          out_specs=pl.BlockSpec((BM, D), lambda s, f, m: (m[0, s], 0)),
      ),
      out_shape=jax.ShapeDtypeStruct((NK, D), jnp.float32),
      compiler_params=pltpu.CompilerParams(
          dimension_semantics=("arbitrary", "arbitrary")),
  )(meta, gathered, W1, b1, W2, b2)

  out_perm = out_sorted[inv]
  out = (out_perm.reshape(N, K, D) * w.reshape(N, K, 1)).sum(axis=1) + attn_out
  return out.reshape(B, T, D)


# speedup vs baseline: 2.0151x; 2.0151x over previous
import functools

import jax
import jax.numpy as jnp
from jax import lax
from jax.experimental import pallas as pl
from jax.experimental.pallas import tpu as pltpu

H = 16
K = 2
QB = 512
BM = 256
FC = 1024
RB = 256


def _ln(x, scale, bias):
  mu = jnp.mean(x, -1, keepdims=True)
  var = jnp.mean((x - mu) ** 2, -1, keepdims=True)
  return (x - mu) * lax.rsqrt(var + 1e-5) * scale + bias


def _ln_qkv_body(x_ref, s_ref, b_ref, w_ref, bias_ref, o_ref):
  h = _ln(x_ref[...], s_ref[...], b_ref[...])
  o_ref[...] = (
      jnp.dot(h, w_ref[...], preferred_element_type=jnp.float32) + bias_ref[...]
  )


def _attn_body(q_ref, k_ref, v_ref, o_ref, *, T, dh):
  i = pl.program_id(2)
  q = q_ref[0]
  k = k_ref[0]
  v = v_ref[0]
  s = lax.dot_general(q, k, (((1,), (1,)), ((), ())),
                      preferred_element_type=jnp.float32)
  s = s * (1.0 / jnp.sqrt(jnp.float32(dh)))
  qrow = i * QB + lax.broadcasted_iota(jnp.int32, (QB, T), 0)
  kcol = lax.broadcasted_iota(jnp.int32, (QB, T), 1)
  s = jnp.where(qrow >= kcol, s, jnp.float32(-1e9))
  m = jnp.max(s, -1, keepdims=True)
  p = jnp.exp(s - m)
  p = p / jnp.sum(p, -1, keepdims=True)
  o_ref[0] = jnp.dot(p, v, preferred_element_type=jnp.float32)


def _proj_ln2_body(a_ref, x_ref, wo_ref, bo_ref, s2_ref, b2_ref, wg_ref,
                   ao_ref, mi_ref, lg_ref):
  a = (jnp.dot(a_ref[...], wo_ref[...], preferred_element_type=jnp.float32)
       + bo_ref[...] + x_ref[...])
  ao_ref[...] = a
  h = _ln(a, s2_ref[...], b2_ref[...])
  mi_ref[...] = h
  lg_ref[...] = jnp.dot(h, wg_ref[...], preferred_element_type=jnp.float32)


def _moe_body(meta_ref, x_ref, w1_ref, b1_ref, w2_ref, b2_ref, o_ref):
  s = pl.program_id(0)
  f = pl.program_id(1)
  blk = meta_ref[0, s]
  e = meta_ref[1, s]
  first = meta_ref[2, s]
  active = meta_ref[3, s]

  @pl.when((first == 1) & (f == 0))
  def _():
    o_ref[...] = jnp.zeros_like(o_ref)

  @pl.when(active == 1)
  def _():
    xb = x_ref[...]
    h = jax.nn.gelu(
        jnp.dot(xb, w1_ref[0], preferred_element_type=jnp.float32)
        + b1_ref[0])
    c = jnp.dot(h, w2_ref[0], preferred_element_type=jnp.float32)
    c = c + jnp.where(f == 0, 1.0, 0.0) * b2_ref[0]
    start = meta_ref[4, e]
    end = meta_ref[4, e + 1]
    row = blk * BM + lax.broadcasted_iota(jnp.int32, (BM, 1), 0)
    mask = (row >= start) & (row < end)
    o_ref[...] += jnp.where(mask, c, 0.0)


def kernel(x, ln1_scale, ln1_bias, ln2_scale, ln2_bias, Wqkv, bqkv, Wo, bo,
           Wg, W1, b1, W2, b2):
  B, T, D = x.shape
  E, _, F = W1.shape
  dh = D // H
  N = B * T
  NK = N * K
  NB = NK // BM
  S = NB + E - 1
  NF = F // FC

  x2d = x.reshape(N, D)
  ln1s = ln1_scale.reshape(1, D)
  ln1b = ln1_bias.reshape(1, D)
  ln2s = ln2_scale.reshape(1, D)
  ln2b = ln2_bias.reshape(1, D)

  qkv = pl.pallas_call(
      _ln_qkv_body,
      grid=(N // RB,),
      in_specs=[
          pl.BlockSpec((RB, D), lambda i: (i, 0)),
          pl.BlockSpec((1, D), lambda i: (0, 0)),
          pl.BlockSpec((1, D), lambda i: (0, 0)),
          pl.BlockSpec((D, 3 * D), lambda i: (0, 0)),
          pl.BlockSpec((1, 3 * D), lambda i: (0, 0)),
      ],
      out_specs=pl.BlockSpec((RB, 3 * D), lambda i: (i, 0)),
      out_shape=jax.ShapeDtypeStruct((N, 3 * D), jnp.float32),
  )(x2d, ln1s, ln1b, Wqkv, bqkv.reshape(1, 3 * D))

  nq = T // QB
  qkv_t = qkv.reshape(N, 3 * H, dh).transpose(1, 0, 2)
  attn_t = pl.pallas_call(
      functools.partial(_attn_body, T=T, dh=dh),
      grid=(B, H, nq),
      in_specs=[
          pl.BlockSpec((1, QB, dh), lambda b, h, i: (h, b * (T // QB) + i, 0)),
          pl.BlockSpec((1, T, dh), lambda b, h, i: (H + h, b, 0)),
          pl.BlockSpec((1, T, dh), lambda b, h, i: (2 * H + h, b, 0)),
      ],
      out_specs=pl.BlockSpec(
          (1, QB, dh), lambda b, h, i: (h, b * (T // QB) + i, 0)),
      out_shape=jax.ShapeDtypeStruct((H, N, dh), jnp.float32),
      compiler_params=pltpu.CompilerParams(
          dimension_semantics=("arbitrary", "arbitrary", "arbitrary")),
  )(qkv_t, qkv_t, qkv_t)
  attn = attn_t.transpose(1, 0, 2).reshape(N, D)

  EP = 128
  wg_p = jnp.zeros((D, EP), jnp.float32).at[:, :E].set(Wg)
  attn_out, moe_in, logits_p = pl.pallas_call(
      _proj_ln2_body,
      grid=(N // RB,),
      in_specs=[
          pl.BlockSpec((RB, D), lambda i: (i, 0)),
          pl.BlockSpec((RB, D), lambda i: (i, 0)),
          pl.BlockSpec((D, D), lambda i: (0, 0)),
          pl.BlockSpec((1, D), lambda i: (0, 0)),
          pl.BlockSpec((1, D), lambda i: (0, 0)),
          pl.BlockSpec((1, D), lambda i: (0, 0)),
          pl.BlockSpec((D, EP), lambda i: (0, 0)),
      ],
      out_specs=[
          pl.BlockSpec((RB, D), lambda i: (i, 0)),
          pl.BlockSpec((RB, D), lambda i: (i, 0)),
          pl.BlockSpec((RB, EP), lambda i: (i, 0)),
      ],
      out_shape=[
          jax.ShapeDtypeStruct((N, D), jnp.float32),
          jax.ShapeDtypeStruct((N, D), jnp.float32),
          jax.ShapeDtypeStruct((N, EP), jnp.float32),
      ],
  )(attn, x2d, Wo, bo.reshape(1, D), ln2s, ln2b, wg_p)

  logits = logits_p[:, :E]

  topv, topi = lax.top_k(logits, K)
  w = jax.nn.softmax(topv, axis=-1)
  flat_e = topi.reshape(-1).astype(jnp.int32)
  onehot = (flat_e[:, None] == jnp.arange(E, dtype=jnp.int32)).astype(jnp.int32)
  incl = jnp.cumsum(onehot, axis=0)
  counts = incl[-1]
  offsets = jnp.concatenate(
      [jnp.zeros((1,), jnp.int32), jnp.cumsum(counts).astype(jnp.int32)])
  rank = jnp.take_along_axis(incl, flat_e[:, None], 1)[:, 0] - 1
  inv = offsets[flat_e] + rank
  order = jnp.zeros((NK,), jnp.int32).at[inv].set(
      jnp.arange(NK, dtype=jnp.int32))
  tok_order = order // K

  start_e = offsets[:E]
  end_e = offsets[1:]
  nonempty = counts > 0
  firstb = jnp.where(nonempty, start_e // BM, 0)
  lastb = jnp.where(nonempty, (end_e - 1) // BM, -1)
  nb_e = jnp.where(nonempty, lastb - firstb + 1, 0).astype(jnp.int32)
  pos = jnp.concatenate(
      [jnp.zeros((1,), jnp.int32), jnp.cumsum(nb_e).astype(jnp.int32)])
  P = pos[E]
  pair_block = jnp.full((S + 1,), NB - 1, jnp.int32)
  pair_expert = jnp.zeros((S + 1,), jnp.int32)
  jblk = jnp.arange(NB, dtype=jnp.int32)
  for e in range(E):
    slots = jnp.where(jblk < nb_e[e], pos[e] + jblk, S)
    pair_block = pair_block.at[slots].set(firstb[e] + jblk)
    pair_expert = pair_expert.at[slots].set(e)
  pair_block = pair_block.at[S - 1].set(
      jnp.where(P < S, NB - 1, pair_block[S - 1]))
  pair_block = pair_block[:S]
  pair_expert = pair_expert[:S]
  pair_block = jnp.where(jnp.arange(S) < P, pair_block, NB - 1)
  first_flag = jnp.concatenate(
      [jnp.ones((1,), jnp.int32),
       (pair_block[1:] != pair_block[:-1]).astype(jnp.int32)])
  active_flag = (jnp.arange(S, dtype=jnp.int32) < P).astype(jnp.int32)
  meta = jnp.zeros((5, 64), jnp.int32)
  meta = meta.at[0, :S].set(pair_block)
  meta = meta.at[1, :S].set(pair_expert)
  meta = meta.at[2, :S].set(first_flag)
  meta = meta.at[3, :S].set(active_flag)
  meta = meta.at[4, : E + 1].set(offsets)

  gathered = moe_in[tok_order]

  out_sorted = pl.pallas_call(
      _moe_body,
      grid_spec=pltpu.PrefetchScalarGridSpec(
          num_scalar_prefetch=1,
          grid=(S, NF),
          in_specs=[
              pl.BlockSpec((BM, D), lambda s, f, m: (m[0, s], 0)),
              pl.BlockSpec((1, D, FC), lambda s, f, m: (m[1, s], 0, f)),
              pl.BlockSpec((1, 1, FC), lambda s, f, m: (m[1, s], 0, f)),
              pl.BlockSpec((1, FC, D), lambda s, f, m: (m[1, s], f, 0)),
              pl.BlockSpec((1, 1, D), lambda s, f, m: (m[1, s], 0, 0)),
          ],
          out_specs=pl.BlockSpec((BM, D), lambda s, f, m: (m[0, s], 0)),
      ),
      out_shape=jax.ShapeDtypeStruct((NK, D), jnp.float32),
      compiler_params=pltpu.CompilerParams(
          dimension_semantics=("arbitrary", "arbitrary")),
  )(meta, gathered, W1, b1.reshape(E, 1, F), W2, b2.reshape(E, 1, D))

  out_perm = out_sorted[inv]
  out = (out_perm.reshape(N, K, D) * w.reshape(N, K, 1)).sum(axis=1) + attn_out
  return out.reshape(B, T, D)
